# baseline (speedup 1.0000x reference)
import functools

import jax
import jax.numpy as jnp
from jax import lax
from jax.experimental import pallas as pl
from jax.experimental.pallas import tpu as pltpu
from jax.experimental.pallas import tpu_sc as plsc

D = 128
N = 10000
E = 320000

NC = 2
NS = 16
L = 16
NW = NC * NS
EPW = E // NW
C = 40
C2 = C // 2
CH = EPW // C
NACC = 10240
NPW = NACC // NS
BN = 1000
NBLK = N // BN


def _edge_phase(x, row3, col3, edge_attr):
    mesh = plsc.VectorSubcoreMesh(core_axis_name="c", subcore_axis_name="s")

    @functools.partial(
        pl.kernel,
        mesh=mesh,
        out_type=jax.ShapeDtypeStruct((2 * NACC, D), jnp.float32),
        scratch_types=[
            pltpu.VMEM_SHARED((NACC, D), jnp.float32),
            pltpu.VMEM((C,), jnp.int32),
            pltpu.VMEM((C,), jnp.int32),
            pltpu.VMEM((C,), jnp.int32),
            pltpu.VMEM((C,), jnp.int32),
            pltpu.VMEM((C,), jnp.int32),
            pltpu.VMEM((C,), jnp.int32),
            pltpu.VMEM((C, D), jnp.float32),
            pltpu.VMEM((C, D), jnp.float32),
            pltpu.VMEM((C, D), jnp.float32),
            pltpu.VMEM((C, D), jnp.float32),
            pltpu.VMEM((C, D), jnp.float32),
            pltpu.VMEM((C, D), jnp.float32),
        ] + [pltpu.SemaphoreType.DMA] * 15,
    )
    def k(x_hbm, row_hbm, col_hbm, ea_hbm, out_hbm, acc_sh,
          rb0, rb1, rb2, cb0, cb1, cb2, xg0, xg1, xg2, ea0, ea1, ea2,
          sr0, sr1, sr2, sc0, sc1, sc2, sg0, sg1, sg2,
          se0, se1, se2, ss0, ss1, ss2):
        c = lax.axis_index("c")
        s = lax.axis_index("s")
        wid = c * NS + s
        ebase = wid * EPW
        rb, cb = (rb0, rb1, rb2), (cb0, cb1, cb2)
        xg, ea = (xg0, xg1, xg2), (ea0, ea1, ea2)
        sr, sc, sg = (sr0, sr1, sr2), (sc0, sc1, sc2), (sg0, sg1, sg2)
        se, ssc = (se0, se1, se2), (ss0, ss1, ss2)

        def zrow(i, _):
            for j in range(D // L):
                xg0[i, pl.ds(j * L, L)] = jnp.zeros((L,), jnp.float32)
            return 0

        lax.fori_loop(0, C, zrow, 0)
        for r in range(NPW // C):
            pltpu.sync_copy(xg0, acc_sh.at[pl.ds(s * NPW + r * C, C)])
        plsc.subcore_barrier()

        def gather_cp(kk, b):
            return pltpu.make_async_copy(x_hbm.at[cb[b]], xg[b], sg[b])

        def ea_cp(kk, b):
            return pltpu.make_async_copy(
                ea_hbm.at[pl.ds(ebase + kk * C, C)], ea[b], se[b])

        def row_cp(kk, b):
            return pltpu.make_async_copy(
                row_hbm.at[pl.ds(ebase + kk * C, C)], rb[b], sr[b])

        def col_cp(kk, b):
            return pltpu.make_async_copy(
                col_hbm.at[pl.ds(ebase + kk * C, C)], cb[b], sc[b])

        def scat_cp(b):
            return pltpu.make_async_copy(xg[b], acc_sh.at[rb[b]], ssc[b])

        def compute_half(b, h):
            def rowfn(i, _):
                for u in range(4):
                    r = h * C2 + 4 * i + u
                    for j in range(D // L):
                        sl = pl.ds(j * L, L)
                        xg[b][r, sl] = jnp.maximum(
                            xg[b][r, sl] + ea[b][r, sl], 0.0)
                return 0

            lax.fori_loop(0, C2 // 4, rowfn, 0)

        def step(kk, b, tail=None):
            b1 = (b + 1) % 3
            b2 = (b + 2) % 3
            gather_cp(kk, b).wait()
            ea_cp(kk, b).wait()
            if tail is None or tail <= CH - 4:
                col_cp(kk + 3, b).start()
            if tail is None or tail <= CH - 3:
                ea_cp(kk + 2, b2).start()
            if tail is None or tail <= CH - 2:
                row_cp(kk + 1, b1).start()
            compute_half(b, 0)

            @pl.when(kk >= 1)
            def _():
                scat_cp(b2).wait()

            if tail is None or tail <= CH - 3:
                col_cp(kk + 2, b2).wait()
                gather_cp(kk + 2, b2).start()
            compute_half(b, 1)
            row_cp(kk, b).wait()
            scat_cp(b).start(add=True)

        pltpu.sync_copy(col_hbm.at[pl.ds(ebase, C)], cb0)
        pltpu.sync_copy(col_hbm.at[pl.ds(ebase + C, C)], cb1)
        gather_cp(0, 0).start()
        gather_cp(1, 1).start()
        ea_cp(0, 0).start()
        ea_cp(1, 1).start()
        col_cp(2, 2).start()
        row_cp(0, 0).start()

        NT = ((CH - 4) // 3) * 3

        def triple(p, _):
            step(3 * p, 0)
            step(3 * p + 1, 1)
            step(3 * p + 2, 2)
            return 0

        lax.fori_loop(0, NT // 3, triple, 0)
        for kk in range(NT, CH):
            step(kk, kk % 3, tail=kk)
        scat_cp((CH - 1) % 3).wait()
        plsc.subcore_barrier()
        pltpu.sync_copy(acc_sh.at[pl.ds(s * NPW, NPW)],
                        out_hbm.at[pl.ds(c * NACC + s * NPW, NPW)])

    return k(x, row3, col3, edge_attr)


def _mlp_body(scale_ref, x_ref, p0_ref, p1_ref, W1_ref, b1_ref, g_ref,
              be_ref, W2_ref, b2_ref, o_ref):
    acc = p0_ref[0] + p1_ref[0]
    h = scale_ref[0, 0] * x_ref[...] + jnp.maximum(acc, 0.0)
    h1 = jnp.dot(h, W1_ref[...], preferred_element_type=jnp.float32)
    h1 = h1 + b1_ref[...]
    mu = jnp.mean(h1, axis=-1, keepdims=True)
    d = h1 - mu
    var = jnp.mean(d * d, axis=-1, keepdims=True)
    h1n = d * lax.rsqrt(var + 1e-5) * g_ref[...] + be_ref[...]
    o = jnp.dot(jnp.maximum(h1n, 0.0), W2_ref[...],
                preferred_element_type=jnp.float32)
    o_ref[...] = o + b2_ref[...]


def _node_phase(x, partials2, W1, b1, gamma, beta, W2, b2, eps_p):
    scale = (1.0 + eps_p).reshape(1, 1)
    return pl.pallas_call(
        _mlp_body,
        grid=(NBLK,),
        in_specs=[
            pl.BlockSpec((1, 1), lambda i: (0, 0)),
            pl.BlockSpec((BN, D), lambda i: (i, 0)),
            pl.BlockSpec((1, BN, D), lambda i: (0, i, 0)),
            pl.BlockSpec((1, BN, D), lambda i: (1, i, 0)),
            pl.BlockSpec((D, 2 * D), lambda i: (0, 0)),
            pl.BlockSpec((1, 2 * D), lambda i: (0, 0)),
            pl.BlockSpec((1, 2 * D), lambda i: (0, 0)),
            pl.BlockSpec((1, 2 * D), lambda i: (0, 0)),
            pl.BlockSpec((2 * D, D), lambda i: (0, 0)),
            pl.BlockSpec((1, D), lambda i: (0, 0)),
        ],
        out_specs=pl.BlockSpec((BN, D), lambda i: (i, 0)),
        out_shape=jax.ShapeDtypeStruct((N, D), jnp.float32),
    )(scale, x, partials2, partials2, W1, b1.reshape(1, -1),
      gamma.reshape(1, -1), beta.reshape(1, -1), W2, b2.reshape(1, -1))


def kernel(x, edge_index, edge_attr, W1, b1, gamma, beta, W2, b2, eps_p):
    partials = _edge_phase(x, edge_index[0], edge_index[1], edge_attr)
    partials2 = partials.reshape(2, NACC, D)
    return _node_phase(x, partials2, W1, b1, gamma, beta, W2, b2, eps_p)


# speedup vs baseline: 1.0056x; 1.0056x over previous
import functools

import jax
import jax.numpy as jnp
from jax import lax
from jax.experimental import pallas as pl
from jax.experimental.pallas import tpu as pltpu
from jax.experimental.pallas import tpu_sc as plsc

D = 128
N = 10000
E = 320000

NC = 2
NS = 16
L = 16
NW = NC * NS
EPW = E // NW
C = 40
C2 = C // 2
CH = EPW // C
NACC = 10240
NPW = NACC // NS
BN = 1000
NBLK = N // BN


def _edge_phase(x, row3, col3, edge_attr):
    mesh = plsc.VectorSubcoreMesh(core_axis_name="c", subcore_axis_name="s")

    @functools.partial(
        pl.kernel,
        mesh=mesh,
        out_type=jax.ShapeDtypeStruct((2 * NACC, D), jnp.float32),
        scratch_types=[
            pltpu.VMEM_SHARED((NACC, D), jnp.float32),
            pltpu.VMEM((C,), jnp.int32),
            pltpu.VMEM((C,), jnp.int32),
            pltpu.VMEM((C,), jnp.int32),
            pltpu.VMEM((C,), jnp.int32),
            pltpu.VMEM((C,), jnp.int32),
            pltpu.VMEM((C,), jnp.int32),
            pltpu.VMEM((C, D), jnp.float32),
            pltpu.VMEM((C, D), jnp.float32),
            pltpu.VMEM((C, D), jnp.float32),
            pltpu.VMEM((C, D), jnp.float32),
            pltpu.VMEM((C, D), jnp.float32),
            pltpu.VMEM((C, D), jnp.float32),
        ] + [pltpu.SemaphoreType.DMA] * 15,
    )
    def k(x_hbm, row_hbm, col_hbm, ea_hbm, out_hbm, acc_sh,
          rb0, rb1, rb2, cb0, cb1, cb2, xg0, xg1, xg2, ea0, ea1, ea2,
          sr0, sr1, sr2, sc0, sc1, sc2, sg0, sg1, sg2,
          se0, se1, se2, ss0, ss1, ss2):
        c = lax.axis_index("c")
        s = lax.axis_index("s")
        wid = c * NS + s
        ebase = wid * EPW
        rb, cb = (rb0, rb1, rb2), (cb0, cb1, cb2)
        xg, ea = (xg0, xg1, xg2), (ea0, ea1, ea2)
        sr, sc, sg = (sr0, sr1, sr2), (sc0, sc1, sc2), (sg0, sg1, sg2)
        se, ssc = (se0, se1, se2), (ss0, ss1, ss2)

        def zrow(i, _):
            for j in range(D // L):
                xg0[i, pl.ds(j * L, L)] = jnp.zeros((L,), jnp.float32)
            return 0

        lax.fori_loop(0, C, zrow, 0)
        for r in range(NPW // C):
            pltpu.sync_copy(xg0, acc_sh.at[pl.ds(s * NPW + r * C, C)])
        plsc.subcore_barrier()

        def gather_cp(kk, b):
            return pltpu.make_async_copy(x_hbm.at[cb[b]], xg[b], sg[b])

        def ea_cp(kk, b):
            return pltpu.make_async_copy(
                ea_hbm.at[pl.ds(ebase + kk * C, C)], ea[b], se[b])

        def row_cp(kk, b):
            return pltpu.make_async_copy(
                row_hbm.at[pl.ds(ebase + kk * C, C)], rb[b], sr[b])

        def col_cp(kk, b):
            return pltpu.make_async_copy(
                col_hbm.at[pl.ds(ebase + kk * C, C)], cb[b], sc[b])

        def scat_cp(b):
            return pltpu.make_async_copy(xg[b], acc_sh.at[rb[b]], ssc[b])

        def compute_half(b, h):
            def rowfn(i, _):
                for u in range(2):
                    r = h * C2 + 2 * i + u
                    for j in range(D // L):
                        sl = pl.ds(j * L, L)
                        xg[b][r, sl] = jnp.maximum(
                            xg[b][r, sl] + ea[b][r, sl], 0.0)
                return 0

            lax.fori_loop(0, C2 // 2, rowfn, 0)

        def step(kk, b, tail=None):
            b1 = (b + 1) % 3
            b2 = (b + 2) % 3
            gather_cp(kk, b).wait()
            ea_cp(kk, b).wait()
            if tail is None or tail <= CH - 4:
                col_cp(kk + 3, b).start()
            if tail is None or tail <= CH - 3:
                ea_cp(kk + 2, b2).start()
            if tail is None or tail <= CH - 2:
                row_cp(kk + 1, b1).start()
            compute_half(b, 0)

            @pl.when(kk >= 1)
            def _():
                scat_cp(b2).wait()

            if tail is None or tail <= CH - 3:
                col_cp(kk + 2, b2).wait()
                gather_cp(kk + 2, b2).start()
            compute_half(b, 1)
            row_cp(kk, b).wait()
            scat_cp(b).start(add=True)

        pltpu.sync_copy(col_hbm.at[pl.ds(ebase, C)], cb0)
        pltpu.sync_copy(col_hbm.at[pl.ds(ebase + C, C)], cb1)
        gather_cp(0, 0).start()
        gather_cp(1, 1).start()
        ea_cp(0, 0).start()
        ea_cp(1, 1).start()
        col_cp(2, 2).start()
        row_cp(0, 0).start()

        NT = ((CH - 4) // 3) * 3

        def triple(p, _):
            step(3 * p, 0)
            step(3 * p + 1, 1)
            step(3 * p + 2, 2)
            return 0

        lax.fori_loop(0, NT // 3, triple, 0)
        for kk in range(NT, CH):
            step(kk, kk % 3, tail=kk)
        scat_cp((CH - 1) % 3).wait()
        plsc.subcore_barrier()
        pltpu.sync_copy(acc_sh.at[pl.ds(s * NPW, NPW)],
                        out_hbm.at[pl.ds(c * NACC + s * NPW, NPW)])

    return k(x, row3, col3, edge_attr)


def _mlp_body(scale_ref, x_ref, p0_ref, p1_ref, W1_ref, b1_ref, g_ref,
              be_ref, W2_ref, b2_ref, o_ref):
    acc = p0_ref[0] + p1_ref[0]
    h = scale_ref[0, 0] * x_ref[...] + jnp.maximum(acc, 0.0)
    h1 = jnp.dot(h, W1_ref[...], preferred_element_type=jnp.float32)
    h1 = h1 + b1_ref[...]
    mu = jnp.mean(h1, axis=-1, keepdims=True)
    d = h1 - mu
    var = jnp.mean(d * d, axis=-1, keepdims=True)
    h1n = d * lax.rsqrt(var + 1e-5) * g_ref[...] + be_ref[...]
    o = jnp.dot(jnp.maximum(h1n, 0.0), W2_ref[...],
                preferred_element_type=jnp.float32)
    o_ref[...] = o + b2_ref[...]


def _node_phase(x, partials2, W1, b1, gamma, beta, W2, b2, eps_p):
    scale = (1.0 + eps_p).reshape(1, 1)
    return pl.pallas_call(
        _mlp_body,
        grid=(NBLK,),
        in_specs=[
            pl.BlockSpec((1, 1), lambda i: (0, 0)),
            pl.BlockSpec((BN, D), lambda i: (i, 0)),
            pl.BlockSpec((1, BN, D), lambda i: (0, i, 0)),
            pl.BlockSpec((1, BN, D), lambda i: (1, i, 0)),
            pl.BlockSpec((D, 2 * D), lambda i: (0, 0)),
            pl.BlockSpec((1, 2 * D), lambda i: (0, 0)),
            pl.BlockSpec((1, 2 * D), lambda i: (0, 0)),
            pl.BlockSpec((1, 2 * D), lambda i: (0, 0)),
            pl.BlockSpec((2 * D, D), lambda i: (0, 0)),
            pl.BlockSpec((1, D), lambda i: (0, 0)),
        ],
        out_specs=pl.BlockSpec((BN, D), lambda i: (i, 0)),
        out_shape=jax.ShapeDtypeStruct((N, D), jnp.float32),
    )(scale, x, partials2, partials2, W1, b1.reshape(1, -1),
      gamma.reshape(1, -1), beta.reshape(1, -1), W2, b2.reshape(1, -1))


def kernel(x, edge_index, edge_attr, W1, b1, gamma, beta, W2, b2, eps_p):
    partials = _edge_phase(x, edge_index[0], edge_index[1], edge_attr)
    partials2 = partials.reshape(2, NACC, D)
    return _node_phase(x, partials2, W1, b1, gamma, beta, W2, b2, eps_p)


# speedup vs baseline: 1.0157x; 1.0100x over previous
import functools

import jax
import jax.numpy as jnp
from jax import lax
from jax.experimental import pallas as pl
from jax.experimental.pallas import tpu as pltpu
from jax.experimental.pallas import tpu_sc as plsc

D = 128
N = 10000
E = 320000

NC = 2
NS = 16
L = 16
NW = NC * NS
EPW = E // NW
C = 40
C2 = C // 2
CH = EPW // C
NACC = 10240
NPW = NACC // NS
BN = 2000
NBLK = N // BN


def _edge_phase(x, row3, col3, edge_attr):
    mesh = plsc.VectorSubcoreMesh(core_axis_name="c", subcore_axis_name="s")

    @functools.partial(
        pl.kernel,
        mesh=mesh,
        out_type=jax.ShapeDtypeStruct((2 * NACC, D), jnp.float32),
        scratch_types=[
            pltpu.VMEM_SHARED((NACC, D), jnp.float32),
            pltpu.VMEM((C,), jnp.int32),
            pltpu.VMEM((C,), jnp.int32),
            pltpu.VMEM((C,), jnp.int32),
            pltpu.VMEM((C,), jnp.int32),
            pltpu.VMEM((C,), jnp.int32),
            pltpu.VMEM((C,), jnp.int32),
            pltpu.VMEM((C, D), jnp.float32),
            pltpu.VMEM((C, D), jnp.float32),
            pltpu.VMEM((C, D), jnp.float32),
            pltpu.VMEM((C, D), jnp.float32),
            pltpu.VMEM((C, D), jnp.float32),
            pltpu.VMEM((C, D), jnp.float32),
        ] + [pltpu.SemaphoreType.DMA] * 15,
    )
    def k(x_hbm, row_hbm, col_hbm, ea_hbm, out_hbm, acc_sh,
          rb0, rb1, rb2, cb0, cb1, cb2, xg0, xg1, xg2, ea0, ea1, ea2,
          sr0, sr1, sr2, sc0, sc1, sc2, sg0, sg1, sg2,
          se0, se1, se2, ss0, ss1, ss2):
        c = lax.axis_index("c")
        s = lax.axis_index("s")
        wid = c * NS + s
        ebase = wid * EPW
        rb, cb = (rb0, rb1, rb2), (cb0, cb1, cb2)
        xg, ea = (xg0, xg1, xg2), (ea0, ea1, ea2)
        sr, sc, sg = (sr0, sr1, sr2), (sc0, sc1, sc2), (sg0, sg1, sg2)
        se, ssc = (se0, se1, se2), (ss0, ss1, ss2)

        def zrow(i, _):
            for j in range(D // L):
                xg0[i, pl.ds(j * L, L)] = jnp.zeros((L,), jnp.float32)
            return 0

        lax.fori_loop(0, C, zrow, 0)
        for r in range(NPW // C):
            pltpu.sync_copy(xg0, acc_sh.at[pl.ds(s * NPW + r * C, C)])
        plsc.subcore_barrier()

        def gather_cp(kk, b):
            return pltpu.make_async_copy(x_hbm.at[cb[b]], xg[b], sg[b])

        def ea_cp(kk, b):
            return pltpu.make_async_copy(
                ea_hbm.at[pl.ds(ebase + kk * C, C)], ea[b], se[b])

        def row_cp(kk, b):
            return pltpu.make_async_copy(
                row_hbm.at[pl.ds(ebase + kk * C, C)], rb[b], sr[b])

        def col_cp(kk, b):
            return pltpu.make_async_copy(
                col_hbm.at[pl.ds(ebase + kk * C, C)], cb[b], sc[b])

        def scat_cp(b):
            return pltpu.make_async_copy(xg[b], acc_sh.at[rb[b]], ssc[b])

        def compute_half(b, h):
            def rowfn(i, _):
                for u in range(2):
                    r = h * C2 + 2 * i + u
                    for j in range(D // L):
                        sl = pl.ds(j * L, L)
                        xg[b][r, sl] = jnp.maximum(
                            xg[b][r, sl] + ea[b][r, sl], 0.0)
                return 0

            lax.fori_loop(0, C2 // 2, rowfn, 0)

        def step(kk, b, tail=None):
            b1 = (b + 1) % 3
            b2 = (b + 2) % 3
            gather_cp(kk, b).wait()
            ea_cp(kk, b).wait()
            if tail is None or tail <= CH - 4:
                col_cp(kk + 3, b).start()
            if tail is None or tail <= CH - 3:
                ea_cp(kk + 2, b2).start()
            if tail is None or tail <= CH - 2:
                row_cp(kk + 1, b1).start()
            compute_half(b, 0)

            @pl.when(kk >= 1)
            def _():
                scat_cp(b2).wait()

            if tail is None or tail <= CH - 3:
                col_cp(kk + 2, b2).wait()
                gather_cp(kk + 2, b2).start()
            compute_half(b, 1)
            row_cp(kk, b).wait()
            scat_cp(b).start(add=True)

        pltpu.sync_copy(col_hbm.at[pl.ds(ebase, C)], cb0)
        pltpu.sync_copy(col_hbm.at[pl.ds(ebase + C, C)], cb1)
        gather_cp(0, 0).start()
        gather_cp(1, 1).start()
        ea_cp(0, 0).start()
        ea_cp(1, 1).start()
        col_cp(2, 2).start()
        row_cp(0, 0).start()

        NT = ((CH - 4) // 3) * 3

        def triple(p, _):
            step(3 * p, 0)
            step(3 * p + 1, 1)
            step(3 * p + 2, 2)
            return 0

        lax.fori_loop(0, NT // 3, triple, 0)
        for kk in range(NT, CH):
            step(kk, kk % 3, tail=kk)
        scat_cp((CH - 1) % 3).wait()
        plsc.subcore_barrier()
        pltpu.sync_copy(acc_sh.at[pl.ds(s * NPW, NPW)],
                        out_hbm.at[pl.ds(c * NACC + s * NPW, NPW)])

    return k(x, row3, col3, edge_attr)


def _mlp_body(scale_ref, x_ref, p0_ref, p1_ref, W1_ref, b1_ref, g_ref,
              be_ref, W2_ref, b2_ref, o_ref):
    acc = p0_ref[0] + p1_ref[0]
    h = scale_ref[0, 0] * x_ref[...] + jnp.maximum(acc, 0.0)
    h1 = jnp.dot(h, W1_ref[...], preferred_element_type=jnp.float32)
    h1 = h1 + b1_ref[...]
    mu = jnp.mean(h1, axis=-1, keepdims=True)
    d = h1 - mu
    var = jnp.mean(d * d, axis=-1, keepdims=True)
    h1n = d * lax.rsqrt(var + 1e-5) * g_ref[...] + be_ref[...]
    o = jnp.dot(jnp.maximum(h1n, 0.0), W2_ref[...],
                preferred_element_type=jnp.float32)
    o_ref[...] = o + b2_ref[...]


def _node_phase(x, partials2, W1, b1, gamma, beta, W2, b2, eps_p):
    scale = (1.0 + eps_p).reshape(1, 1)
    return pl.pallas_call(
        _mlp_body,
        grid=(NBLK,),
        in_specs=[
            pl.BlockSpec((1, 1), lambda i: (0, 0)),
            pl.BlockSpec((BN, D), lambda i: (i, 0)),
            pl.BlockSpec((1, BN, D), lambda i: (0, i, 0)),
            pl.BlockSpec((1, BN, D), lambda i: (1, i, 0)),
            pl.BlockSpec((D, 2 * D), lambda i: (0, 0)),
            pl.BlockSpec((1, 2 * D), lambda i: (0, 0)),
            pl.BlockSpec((1, 2 * D), lambda i: (0, 0)),
            pl.BlockSpec((1, 2 * D), lambda i: (0, 0)),
            pl.BlockSpec((2 * D, D), lambda i: (0, 0)),
            pl.BlockSpec((1, D), lambda i: (0, 0)),
        ],
        out_specs=pl.BlockSpec((BN, D), lambda i: (i, 0)),
        out_shape=jax.ShapeDtypeStruct((N, D), jnp.float32),
    )(scale, x, partials2, partials2, W1, b1.reshape(1, -1),
      gamma.reshape(1, -1), beta.reshape(1, -1), W2, b2.reshape(1, -1))


def kernel(x, edge_index, edge_attr, W1, b1, gamma, beta, W2, b2, eps_p):
    partials = _edge_phase(x, edge_index[0], edge_index[1], edge_attr)
    partials2 = partials.reshape(2, NACC, D)
    return _node_phase(x, partials2, W1, b1, gamma, beta, W2, b2, eps_p)


# speedup vs baseline: 1.0482x; 1.0320x over previous
import functools

import jax
import jax.numpy as jnp
from jax import lax
from jax.experimental import pallas as pl
from jax.experimental.pallas import tpu as pltpu
from jax.experimental.pallas import tpu_sc as plsc

D = 128
N = 10000
E = 320000

NC = 2
NS = 16
L = 16
NW = NC * NS
EPW = E // NW
C = 40
C2 = C // 2
CH = EPW // C
NACC = 10240
NPW = NACC // NS
BN = 2000
NBLK = N // BN


def _edge_phase(x, row3, col3, edge_attr):
    mesh = plsc.VectorSubcoreMesh(core_axis_name="c", subcore_axis_name="s")

    @functools.partial(
        pl.kernel,
        mesh=mesh,
        out_type=jax.ShapeDtypeStruct((2 * NACC, D), jnp.float32),
        scratch_types=[
            pltpu.VMEM_SHARED((NACC, D), jnp.float32),
            pltpu.VMEM((C,), jnp.int32),
            pltpu.VMEM((C,), jnp.int32),
            pltpu.VMEM((C,), jnp.int32),
            pltpu.VMEM((C,), jnp.int32),
            pltpu.VMEM((C,), jnp.int32),
            pltpu.VMEM((C,), jnp.int32),
            pltpu.VMEM((C, D), jnp.float32),
            pltpu.VMEM((C, D), jnp.float32),
            pltpu.VMEM((C, D), jnp.float32),
            pltpu.VMEM((C, D), jnp.float32),
            pltpu.VMEM((C, D), jnp.float32),
            pltpu.VMEM((C, D), jnp.float32),
        ] + [pltpu.SemaphoreType.DMA] * 15,
    )
    def k(x_hbm, row_hbm, col_hbm, ea_hbm, out_hbm, acc_sh,
          rb0, rb1, rb2, cb0, cb1, cb2, xg0, xg1, xg2, ea0, ea1, ea2,
          sr0, sr1, sr2, sc0, sc1, sc2, sg0, sg1, sg2,
          se0, se1, se2, ss0, ss1, ss2):
        c = lax.axis_index("c")
        s = lax.axis_index("s")
        wid = c * NS + s
        ebase = wid * EPW
        rb, cb = (rb0, rb1, rb2), (cb0, cb1, cb2)
        xg, ea = (xg0, xg1, xg2), (ea0, ea1, ea2)
        sr, sc, sg = (sr0, sr1, sr2), (sc0, sc1, sc2), (sg0, sg1, sg2)
        se, ssc = (se0, se1, se2), (ss0, ss1, ss2)

        def zrow(i, _):
            for j in range(D // L):
                xg0[i, pl.ds(j * L, L)] = jnp.zeros((L,), jnp.float32)
            return 0

        lax.fori_loop(0, C, zrow, 0)
        for r in range(NPW // C):
            pltpu.make_async_copy(
                xg0, acc_sh.at[pl.ds(s * NPW + r * C, C)], ss0).start()
        for r in range(NPW // C):
            pltpu.make_async_copy(
                xg0, acc_sh.at[pl.ds(s * NPW + r * C, C)], ss0).wait()
        plsc.subcore_barrier()

        def gather_cp(kk, b):
            return pltpu.make_async_copy(x_hbm.at[cb[b]], xg[b], sg[b])

        def ea_cp(kk, b):
            return pltpu.make_async_copy(
                ea_hbm.at[pl.ds(ebase + kk * C, C)], ea[b], se[b])

        def row_cp(kk, b):
            return pltpu.make_async_copy(
                row_hbm.at[pl.ds(ebase + kk * C, C)], rb[b], sr[b])

        def col_cp(kk, b):
            return pltpu.make_async_copy(
                col_hbm.at[pl.ds(ebase + kk * C, C)], cb[b], sc[b])

        def scat_cp(b):
            return pltpu.make_async_copy(xg[b], acc_sh.at[rb[b]], ssc[b])

        def compute_half(b, h):
            def rowfn(i, _):
                for u in range(2):
                    r = h * C2 + 2 * i + u
                    for j in range(D // L):
                        sl = pl.ds(j * L, L)
                        xg[b][r, sl] = jnp.maximum(
                            xg[b][r, sl] + ea[b][r, sl], 0.0)
                return 0

            lax.fori_loop(0, C2 // 2, rowfn, 0)

        def step(kk, b, tail=None):
            b1 = (b + 1) % 3
            b2 = (b + 2) % 3
            gather_cp(kk, b).wait()
            ea_cp(kk, b).wait()
            if tail is None or tail <= CH - 4:
                col_cp(kk + 3, b).start()
            if tail is None or tail <= CH - 3:
                ea_cp(kk + 2, b2).start()
            if tail is None or tail <= CH - 2:
                row_cp(kk + 1, b1).start()
            compute_half(b, 0)

            @pl.when(kk >= 1)
            def _():
                scat_cp(b2).wait()

            if tail is None or tail <= CH - 3:
                col_cp(kk + 2, b2).wait()
                gather_cp(kk + 2, b2).start()
            compute_half(b, 1)
            row_cp(kk, b).wait()
            scat_cp(b).start(add=True)

        pltpu.sync_copy(col_hbm.at[pl.ds(ebase, C)], cb0)
        pltpu.sync_copy(col_hbm.at[pl.ds(ebase + C, C)], cb1)
        gather_cp(0, 0).start()
        gather_cp(1, 1).start()
        ea_cp(0, 0).start()
        ea_cp(1, 1).start()
        col_cp(2, 2).start()
        row_cp(0, 0).start()

        NT = ((CH - 4) // 3) * 3

        def triple(p, _):
            step(3 * p, 0)
            step(3 * p + 1, 1)
            step(3 * p + 2, 2)
            return 0

        lax.fori_loop(0, NT // 3, triple, 0)
        for kk in range(NT, CH):
            step(kk, kk % 3, tail=kk)
        scat_cp((CH - 1) % 3).wait()
        plsc.subcore_barrier()
        pltpu.sync_copy(acc_sh.at[pl.ds(s * NPW, NPW)],
                        out_hbm.at[pl.ds(c * NACC + s * NPW, NPW)])

    return k(x, row3, col3, edge_attr)


def _mlp_body(scale_ref, x_ref, p0_ref, p1_ref, W1_ref, b1_ref, g_ref,
              be_ref, W2_ref, b2_ref, o_ref):
    acc = p0_ref[0] + p1_ref[0]
    h = scale_ref[0, 0] * x_ref[...] + jnp.maximum(acc, 0.0)
    h1 = jnp.dot(h, W1_ref[...], preferred_element_type=jnp.float32)
    h1 = h1 + b1_ref[...]
    mu = jnp.mean(h1, axis=-1, keepdims=True)
    d = h1 - mu
    var = jnp.mean(d * d, axis=-1, keepdims=True)
    h1n = d * lax.rsqrt(var + 1e-5) * g_ref[...] + be_ref[...]
    o = jnp.dot(jnp.maximum(h1n, 0.0), W2_ref[...],
                preferred_element_type=jnp.float32)
    o_ref[...] = o + b2_ref[...]


def _node_phase(x, partials2, W1, b1, gamma, beta, W2, b2, eps_p):
    scale = (1.0 + eps_p).reshape(1, 1)
    return pl.pallas_call(
        _mlp_body,
        grid=(NBLK,),
        in_specs=[
            pl.BlockSpec((1, 1), lambda i: (0, 0)),
            pl.BlockSpec((BN, D), lambda i: (i, 0)),
            pl.BlockSpec((1, BN, D), lambda i: (0, i, 0)),
            pl.BlockSpec((1, BN, D), lambda i: (1, i, 0)),
            pl.BlockSpec((D, 2 * D), lambda i: (0, 0)),
            pl.BlockSpec((1, 2 * D), lambda i: (0, 0)),
            pl.BlockSpec((1, 2 * D), lambda i: (0, 0)),
            pl.BlockSpec((1, 2 * D), lambda i: (0, 0)),
            pl.BlockSpec((2 * D, D), lambda i: (0, 0)),
            pl.BlockSpec((1, D), lambda i: (0, 0)),
        ],
        out_specs=pl.BlockSpec((BN, D), lambda i: (i, 0)),
        out_shape=jax.ShapeDtypeStruct((N, D), jnp.float32),
    )(scale, x, partials2, partials2, W1, b1.reshape(1, -1),
      gamma.reshape(1, -1), beta.reshape(1, -1), W2, b2.reshape(1, -1))


def kernel(x, edge_index, edge_attr, W1, b1, gamma, beta, W2, b2, eps_p):
    partials = _edge_phase(x, edge_index[0], edge_index[1], edge_attr)
    partials2 = partials.reshape(2, NACC, D)
    return _node_phase(x, partials2, W1, b1, gamma, beta, W2, b2, eps_p)
